# baseline (speedup 1.0000x reference)
import numpy as np

import jax
import jax.numpy as jnp
from jax.experimental import pallas as pl
from jax.experimental.pallas import tpu as pltpu
from jax.experimental.shard_map import shard_map
from jax.sharding import Mesh, PartitionSpec as P

_SQRT_2_OVER_PI = 0.7978845608028654
_BF = jnp.bfloat16


def _make_body(H, W, C, BT):
    HW = H * W

    def body(x_ref, rowm_ref, colm_ref, w1s_ref, b1_ref, mks_ref, bpf_ref,
             f1w_ref, f1b_ref, f2wt_ref, f2b_ref, bns_ref, bnb_ref, out_ref,
             x3_ref, x5_ref):

        def row_shifted(ai32, dy):
            if dy == 0:
                return ai32
            s = pltpu.roll(ai32, (-dy * W) % HW, axis=1)
            return s & rowm_ref[dy + 2:dy + 3, :]

        def col_shift_add(acc, g, dx):
            s = pltpu.roll(g, (-dx) % HW, axis=1)
            return acc + s * colm_ref[dx + 2:dx + 3, :]

        def to_i32(a_bf):
            return pltpu.bitcast(a_bf, jnp.int32)

        def to_bf(a_i32):
            return pltpu.bitcast(a_i32, _BF)

        for i in range(BT):
            xf = x_ref[i]

            xi = to_i32(xf.astype(_BF))
            for j in range(3):
                x3_ref[j * C:(j + 1) * C, :] = to_bf(row_shifted(xi, j - 1))
            a = jnp.dot(w1s_ref[...], x3_ref[...],
                        preferred_element_type=jnp.float32)
            y1 = a[C:2 * C] + b1_ref[...]
            y1 = col_shift_add(y1, a[:C], -1)
            y1 = col_shift_add(y1, a[2 * C:], 1)

            yi = to_i32(y1.astype(_BF))
            for j in range(5):
                x5_ref[j * C:(j + 1) * C, :] = to_bf(row_shifted(yi, j - 2))
            g = jnp.dot(mks_ref[...], x5_ref[...],
                        preferred_element_type=jnp.float32)
            z = g[2 * C:3 * C] + bpf_ref[...]
            for d in (0, 1, 3, 4):
                z = col_shift_add(z, g[d * C:(d + 1) * C], d - 2)

            gap = jnp.sum(z, axis=1, keepdims=True) * (1.0 / HW)
            h = jnp.maximum(
                jnp.sum(f1w_ref[...] * gap, axis=0, keepdims=True)
                + f1b_ref[...], 0.0)
            gate = jax.nn.sigmoid(
                jnp.sum(f2wt_ref[...] * h, axis=1, keepdims=True)
                + f2b_ref[...])
            zg = z * gate

            inner = _SQRT_2_OVER_PI * (zg + 0.044715 * (zg * zg * zg))
            act = zg * (1.0 + jnp.tanh(inner))
            out_ref[i] = act * bns_ref[...] + bnb_ref[...] + x_ref[i]

    return body


def kernel(x, w1, b1, w3, b3, w5, b5, wp, bp, f1w, f1b, f2w, f2b, bns, bnb):
    B, C, H, W = x.shape
    HW = H * W
    BT = 8 if B % 8 == 0 else (4 if B % 4 == 0 else (2 if B % 2 == 0 else 1))
    x2 = x.reshape(B, C, HW).astype(jnp.float32)

    w1r = w1.reshape(3, 3, C, C)
    w1s = jnp.transpose(w1r, (1, 3, 0, 2)).reshape(3 * C, 3 * C).astype(_BF)
    b1r = b1.reshape(C, 1)

    wp3t = wp[:C, :].T
    wp5t = wp[C:, :].T
    w3p = jnp.zeros((5, 5, C), w3.dtype).at[1:4, 1:4, :].set(w3.reshape(3, 3, C))
    mk4 = (wp5t[None, None] * w5.reshape(5, 5, C)[:, :, None, :]
           + wp3t[None, None] * w3p[:, :, None, :])
    mks = jnp.transpose(mk4, (1, 2, 0, 3)).reshape(5 * C, 5 * C).astype(_BF)
    bpf = (bp + b3 @ wp[:C, :] + b5 @ wp[C:, :]).reshape(C, 1)

    f2wt = f2w.T
    f2br = f2b.reshape(C, 1)
    bns2 = (0.5 * bns).reshape(C, 1)
    bnb2 = bnb.reshape(C, 1)

    dd = jnp.arange(-2, 3)[:, None]
    hh = jnp.arange(H)[None, :]
    ww = jnp.arange(W)[None, :]
    hv = (hh + dd >= 0) & (hh + dd < H)
    wv = (ww + dd >= 0) & (ww + dd < W)
    rowm = (hv.astype(jnp.int32) * jnp.int32(-1))[:, :, None]
    rowm = jnp.broadcast_to(rowm, (5, H, W)).reshape(5, HW)
    colm = jnp.broadcast_to(wv.astype(jnp.float32)[:, None, :],
                            (5, H, W)).reshape(5, HW)

    weights = (rowm, colm, w1s, b1r, mks, bpf, f1w, f1b, f2wt, f2br, bns2, bnb2)

    def const_spec(a):
        nd = a.ndim
        return pl.BlockSpec(a.shape, lambda b, _nd=nd: (0,) * _nd)

    in_specs = [pl.BlockSpec((BT, C, HW), lambda b: (b, 0, 0))]
    in_specs += [const_spec(a) for a in weights]

    def fwd(xs, *ws):
        b_local = xs.shape[0]
        return pl.pallas_call(
            _make_body(H, W, C, BT),
            out_shape=jax.ShapeDtypeStruct((b_local, C, HW), jnp.float32),
            grid_spec=pltpu.PrefetchScalarGridSpec(
                num_scalar_prefetch=0,
                grid=(b_local // BT,),
                in_specs=in_specs,
                out_specs=pl.BlockSpec((BT, C, HW), lambda b: (b, 0, 0)),
                scratch_shapes=[
                    pltpu.VMEM((3 * C, HW), _BF),
                    pltpu.VMEM((5 * C, HW), _BF),
                ],
            ),
            compiler_params=pltpu.CompilerParams(
                dimension_semantics=("parallel",),
                vmem_limit_bytes=100 * 1024 * 1024,
            ),
        )(xs, *ws)

    devs = jax.devices()
    if len(devs) >= 2 and B % (2 * BT) == 0:
        mesh = Mesh(np.asarray(devs[:2]), ("b",))
        fwd_sharded = shard_map(
            fwd, mesh=mesh,
            in_specs=(P("b"),) + (P(),) * len(weights),
            out_specs=P("b"), check_rep=False)
        out2 = fwd_sharded(x2, *weights)
    else:
        out2 = fwd(x2, *weights)
    return out2.reshape(B, C, H, W)


# speedup vs baseline: 2.8071x; 2.8071x over previous
import jax
import jax.numpy as jnp
from jax.experimental import pallas as pl
from jax.experimental.pallas import tpu as pltpu

_SQRT_2_OVER_PI = 0.7978845608028654
_BF = jnp.bfloat16


def _make_body(H, W, C, BT):
    HW = H * W

    def body(x_ref, rowm_ref, colm_ref, w1s_ref, b1_ref, mks_ref, bpf_ref,
             f1w_ref, f1b_ref, f2wt_ref, f2b_ref, bns_ref, bnb_ref, out_ref,
             x3_ref, x5_ref):

        def row_shifted(ai32, dy):
            if dy == 0:
                return ai32
            s = pltpu.roll(ai32, (-dy * W) % HW, axis=1)
            return s & rowm_ref[dy + 2:dy + 3, :]

        def to_i32(a_bf):
            return pltpu.bitcast(a_bf, jnp.int32)

        def to_bf(a_i32):
            return pltpu.bitcast(a_i32, _BF)

        def col_shifted(g_f32, dx):
            s = pltpu.roll(to_i32(g_f32.astype(_BF)), (-dx) % HW, axis=1)
            return to_bf(s & colm_ref[dx + 2:dx + 3, :])

        for i in range(BT):
            xf = x_ref[i]

            xi = to_i32(xf.astype(_BF))
            for j in range(3):
                x3_ref[j * C:(j + 1) * C, :] = to_bf(row_shifted(xi, j - 1))
            a = jnp.dot(w1s_ref[...], x3_ref[...],
                        preferred_element_type=jnp.float32)
            y1 = (a[C:2 * C] + b1_ref[...]
                  + (col_shifted(a[:C], -1)
                     + col_shifted(a[2 * C:], 1)).astype(jnp.float32))

            yi = to_i32(y1.astype(_BF))
            for j in range(5):
                x5_ref[j * C:(j + 1) * C, :] = to_bf(row_shifted(yi, j - 2))
            g = jnp.dot(mks_ref[...], x5_ref[...],
                        preferred_element_type=jnp.float32)
            zs = ((col_shifted(g[:C], -2) + col_shifted(g[C:2 * C], -1))
                  + (col_shifted(g[3 * C:4 * C], 1)
                     + col_shifted(g[4 * C:], 2)))
            z = g[2 * C:3 * C] + bpf_ref[...] + zs.astype(jnp.float32)

            gap = jnp.sum(z, axis=1, keepdims=True) * (1.0 / HW)
            h = jnp.maximum(
                jnp.sum(f1w_ref[...] * gap, axis=0, keepdims=True)
                + f1b_ref[...], 0.0)
            gate = jax.nn.sigmoid(
                jnp.sum(f2wt_ref[...] * h, axis=1, keepdims=True)
                + f2b_ref[...])
            zg = z * gate

            inner = _SQRT_2_OVER_PI * (zg + 0.044715 * (zg * zg * zg))
            act = zg * (1.0 + jnp.tanh(inner))
            out_ref[i] = act * bns_ref[...] + bnb_ref[...] + x_ref[i]

    return body


def kernel(x, w1, b1, w3, b3, w5, b5, wp, bp, f1w, f1b, f2w, f2b, bns, bnb):
    B, C, H, W = x.shape
    HW = H * W
    BT = 8 if B % 8 == 0 else (4 if B % 4 == 0 else (2 if B % 2 == 0 else 1))
    x2 = x.reshape(B, C, HW).astype(jnp.float32)

    w1r = w1.reshape(3, 3, C, C)
    w1s = jnp.transpose(w1r, (1, 3, 0, 2)).reshape(3 * C, 3 * C).astype(_BF)
    b1r = b1.reshape(C, 1)

    wp3t = wp[:C, :].T
    wp5t = wp[C:, :].T
    w3p = jnp.zeros((5, 5, C), w3.dtype).at[1:4, 1:4, :].set(w3.reshape(3, 3, C))
    mk4 = (wp5t[None, None] * w5.reshape(5, 5, C)[:, :, None, :]
           + wp3t[None, None] * w3p[:, :, None, :])
    mks = jnp.transpose(mk4, (1, 2, 0, 3)).reshape(5 * C, 5 * C).astype(_BF)
    bpf = (bp + b3 @ wp[:C, :] + b5 @ wp[C:, :]).reshape(C, 1)

    f2wt = f2w.T
    f2br = f2b.reshape(C, 1)
    bns2 = (0.5 * bns).reshape(C, 1)
    bnb2 = bnb.reshape(C, 1)

    dd = jnp.arange(-2, 3)[:, None]
    hh = jnp.arange(H)[None, :]
    ww = jnp.arange(W)[None, :]
    hv = (hh + dd >= 0) & (hh + dd < H)
    wv = (ww + dd >= 0) & (ww + dd < W)
    rowm = (hv.astype(jnp.int32) * jnp.int32(-1))[:, :, None]
    rowm = jnp.broadcast_to(rowm, (5, H, W)).reshape(5, HW)
    colm = (wv.astype(jnp.int32) * jnp.int32(-1))[:, None, :]
    colm = jnp.broadcast_to(colm, (5, H, W)).reshape(5, HW)

    weights = (rowm, colm, w1s, b1r, mks, bpf, f1w, f1b, f2wt, f2br, bns2, bnb2)

    def const_spec(a):
        nd = a.ndim
        return pl.BlockSpec(a.shape, lambda b, _nd=nd: (0,) * _nd)

    in_specs = [pl.BlockSpec((BT, C, HW), lambda b: (b, 0, 0))]
    in_specs += [const_spec(a) for a in weights]

    out2 = pl.pallas_call(
        _make_body(H, W, C, BT),
        out_shape=jax.ShapeDtypeStruct((B, C, HW), jnp.float32),
        grid_spec=pltpu.PrefetchScalarGridSpec(
            num_scalar_prefetch=0,
            grid=(B // BT,),
            in_specs=in_specs,
            out_specs=pl.BlockSpec((BT, C, HW), lambda b: (b, 0, 0)),
            scratch_shapes=[
                pltpu.VMEM((3 * C, HW), _BF),
                pltpu.VMEM((5 * C, HW), _BF),
            ],
        ),
        compiler_params=pltpu.CompilerParams(
            dimension_semantics=("parallel",),
            vmem_limit_bytes=100 * 1024 * 1024,
        ),
    )(x2, *weights)
    return out2.reshape(B, C, H, W)


# speedup vs baseline: 2.8694x; 1.0222x over previous
import jax
import jax.numpy as jnp
from jax.experimental import pallas as pl
from jax.experimental.pallas import tpu as pltpu

_SQRT_2_OVER_PI = 0.7978845608028654
_BF = jnp.bfloat16


def _make_body(H, W, C, BT):
    HW = H * W

    def body(x_ref, rowm_ref, colm_ref, w1s_ref, b1_ref, mks_ref, bpf_ref,
             f1w_ref, f1b_ref, f2wt_ref, f2b_ref, bns_ref, bnb_ref, out_ref,
             x3_ref, x5_ref):

        def row_shifted(ai32, dy):
            if dy == 0:
                return ai32
            s = pltpu.roll(ai32, (-dy * W) % HW, axis=1)
            return s & rowm_ref[dy + 2:dy + 3, :]

        def to_i32(a_bf):
            return pltpu.bitcast(a_bf, jnp.int32)

        def to_bf(a_i32):
            return pltpu.bitcast(a_i32, _BF)

        def col_shifted(g_f32, dx):
            s = pltpu.roll(to_i32(g_f32.astype(_BF)), (-dx) % HW, axis=1)
            return to_bf(s & colm_ref[dx + 2:dx + 3, :])

        for i in range(BT):
            xf = x_ref[i]

            xi = to_i32(xf.astype(_BF))
            for j in range(3):
                x3_ref[j * C:(j + 1) * C, :] = to_bf(row_shifted(xi, j - 1))
            a = jnp.dot(w1s_ref[...], x3_ref[...],
                        preferred_element_type=jnp.float32)
            y1 = (a[C:2 * C] + b1_ref[...]
                  + (col_shifted(a[:C], -1)
                     + col_shifted(a[2 * C:], 1)).astype(jnp.float32))

            yi = to_i32(y1.astype(_BF))
            for j in range(5):
                x5_ref[j * C:(j + 1) * C, :] = to_bf(row_shifted(yi, j - 2))
            g = jnp.dot(mks_ref[...], x5_ref[...],
                        preferred_element_type=jnp.float32)
            zs = ((col_shifted(g[:C], -2) + col_shifted(g[C:2 * C], -1))
                  + (col_shifted(g[3 * C:4 * C], 1)
                     + col_shifted(g[4 * C:], 2)))
            z = g[2 * C:3 * C] + bpf_ref[...] + zs.astype(jnp.float32)

            gap = jnp.sum(z, axis=1, keepdims=True) * (1.0 / HW)
            h = jnp.maximum(
                jnp.sum(f1w_ref[...] * gap, axis=0, keepdims=True)
                + f1b_ref[...], 0.0)
            gate = jax.nn.sigmoid(
                jnp.sum(f2wt_ref[...] * h, axis=1, keepdims=True)
                + f2b_ref[...])
            zg = z * gate

            inner = _SQRT_2_OVER_PI * (zg + 0.044715 * (zg * zg * zg))
            act = zg * (1.0 + jnp.tanh(inner))
            out_ref[i] = act * bns_ref[...] + bnb_ref[...] + x_ref[i]

    return body


def kernel(x, w1, b1, w3, b3, w5, b5, wp, bp, f1w, f1b, f2w, f2b, bns, bnb):
    B, C, H, W = x.shape
    HW = H * W
    BT = 4 if B % 4 == 0 else (2 if B % 2 == 0 else 1)
    x2 = x.reshape(B, C, HW).astype(jnp.float32)

    w1r = w1.reshape(3, 3, C, C)
    w1s = jnp.transpose(w1r, (1, 3, 0, 2)).reshape(3 * C, 3 * C).astype(_BF)
    b1r = b1.reshape(C, 1)

    wp3t = wp[:C, :].T
    wp5t = wp[C:, :].T
    w3p = jnp.zeros((5, 5, C), w3.dtype).at[1:4, 1:4, :].set(w3.reshape(3, 3, C))
    mk4 = (wp5t[None, None] * w5.reshape(5, 5, C)[:, :, None, :]
           + wp3t[None, None] * w3p[:, :, None, :])
    mks = jnp.transpose(mk4, (1, 2, 0, 3)).reshape(5 * C, 5 * C).astype(_BF)
    bpf = (bp + b3 @ wp[:C, :] + b5 @ wp[C:, :]).reshape(C, 1)

    f2wt = f2w.T
    f2br = f2b.reshape(C, 1)
    bns2 = (0.5 * bns).reshape(C, 1)
    bnb2 = bnb.reshape(C, 1)

    dd = jnp.arange(-2, 3)[:, None]
    hh = jnp.arange(H)[None, :]
    ww = jnp.arange(W)[None, :]
    hv = (hh + dd >= 0) & (hh + dd < H)
    wv = (ww + dd >= 0) & (ww + dd < W)
    rowm = (hv.astype(jnp.int32) * jnp.int32(-1))[:, :, None]
    rowm = jnp.broadcast_to(rowm, (5, H, W)).reshape(5, HW)
    colm = (wv.astype(jnp.int32) * jnp.int32(-1))[:, None, :]
    colm = jnp.broadcast_to(colm, (5, H, W)).reshape(5, HW)

    weights = (rowm, colm, w1s, b1r, mks, bpf, f1w, f1b, f2wt, f2br, bns2, bnb2)

    def const_spec(a):
        nd = a.ndim
        return pl.BlockSpec(a.shape, lambda b, _nd=nd: (0,) * _nd)

    in_specs = [pl.BlockSpec((BT, C, HW), lambda b: (b, 0, 0))]
    in_specs += [const_spec(a) for a in weights]

    out2 = pl.pallas_call(
        _make_body(H, W, C, BT),
        out_shape=jax.ShapeDtypeStruct((B, C, HW), jnp.float32),
        grid_spec=pltpu.PrefetchScalarGridSpec(
            num_scalar_prefetch=0,
            grid=(B // BT,),
            in_specs=in_specs,
            out_specs=pl.BlockSpec((BT, C, HW), lambda b: (b, 0, 0)),
            scratch_shapes=[
                pltpu.VMEM((3 * C, HW), _BF),
                pltpu.VMEM((5 * C, HW), _BF),
            ],
        ),
        compiler_params=pltpu.CompilerParams(
            dimension_semantics=("parallel",),
            vmem_limit_bytes=100 * 1024 * 1024,
        ),
    )(x2, *weights)
    return out2.reshape(B, C, H, W)


# speedup vs baseline: 3.4264x; 1.1941x over previous
import jax
import jax.numpy as jnp
from jax.experimental import pallas as pl
from jax.experimental.pallas import tpu as pltpu

_SQRT_2_OVER_PI = 0.7978845608028654
_BF = jnp.bfloat16


def _make_body(H, W, C, BT):
    HW = H * W

    def body(x_ref, rowm_ref, colm_ref, w1s_ref, b1_ref, mks_ref, bpf_ref,
             f1w_ref, f1b_ref, f2wt_ref, f2b_ref, bns_ref, bnb_ref, out_ref,
             x3_ref, x5_ref):

        def row_shifted(ai32, dy):
            if dy == 0:
                return ai32
            s = pltpu.roll(ai32, (-dy * W) % HW, axis=1)
            return s & rowm_ref[dy + 2:dy + 3, :]

        def to_i32(a_bf):
            return pltpu.bitcast(a_bf, jnp.int32)

        def to_bf(a_i32):
            return pltpu.bitcast(a_i32, _BF)

        def col_shifted(g_f32, dx):
            s = pltpu.roll(to_i32(g_f32.astype(_BF)), (-dx) % HW, axis=1)
            return to_bf(s & colm_ref[dx + 2:dx + 3, :])

        for i in range(BT):
            xi = to_i32(x_ref[i].astype(_BF))
            for j in range(3):
                x3_ref[j * C:(j + 1) * C, i * HW:(i + 1) * HW] = (
                    to_bf(row_shifted(xi, j - 1)))
        a = jnp.dot(w1s_ref[...], x3_ref[...],
                    preferred_element_type=jnp.float32)

        for i in range(BT):
            sl = slice(i * HW, (i + 1) * HW)
            y1 = (a[C:2 * C, sl] + b1_ref[...]
                  + (col_shifted(a[:C, sl], -1)
                     + col_shifted(a[2 * C:, sl], 1)).astype(jnp.float32))
            yi = to_i32(y1.astype(_BF))
            for j in range(5):
                x5_ref[j * C:(j + 1) * C, i * HW:(i + 1) * HW] = (
                    to_bf(row_shifted(yi, j - 2)))
        g = jnp.dot(mks_ref[...], x5_ref[...],
                    preferred_element_type=jnp.float32)

        for i in range(BT):
            sl = slice(i * HW, (i + 1) * HW)
            zs = ((col_shifted(g[:C, sl], -2)
                   + col_shifted(g[C:2 * C, sl], -1))
                  + (col_shifted(g[3 * C:4 * C, sl], 1)
                     + col_shifted(g[4 * C:, sl], 2)))
            z = g[2 * C:3 * C, sl] + bpf_ref[...] + zs.astype(jnp.float32)

            gap = jnp.sum(z, axis=1, keepdims=True) * (1.0 / HW)
            h = jnp.maximum(
                jnp.sum(f1w_ref[...] * gap, axis=0, keepdims=True)
                + f1b_ref[...], 0.0)
            gate = jax.nn.sigmoid(
                jnp.sum(f2wt_ref[...] * h, axis=1, keepdims=True)
                + f2b_ref[...])
            zg = z * gate

            inner = _SQRT_2_OVER_PI * (zg + 0.044715 * (zg * zg * zg))
            act = zg * (1.0 + jnp.tanh(inner))
            out_ref[i] = act * bns_ref[...] + bnb_ref[...] + x_ref[i]

    return body


def kernel(x, w1, b1, w3, b3, w5, b5, wp, bp, f1w, f1b, f2w, f2b, bns, bnb):
    B, C, H, W = x.shape
    HW = H * W
    BT = 4 if B % 4 == 0 else (2 if B % 2 == 0 else 1)
    x2 = x.reshape(B, C, HW).astype(jnp.float32)

    w1r = w1.reshape(3, 3, C, C)
    w1s = jnp.transpose(w1r, (1, 3, 0, 2)).reshape(3 * C, 3 * C).astype(_BF)
    b1r = b1.reshape(C, 1)

    wp3t = wp[:C, :].T
    wp5t = wp[C:, :].T
    w3p = jnp.zeros((5, 5, C), w3.dtype).at[1:4, 1:4, :].set(w3.reshape(3, 3, C))
    mk4 = (wp5t[None, None] * w5.reshape(5, 5, C)[:, :, None, :]
           + wp3t[None, None] * w3p[:, :, None, :])
    mks = jnp.transpose(mk4, (1, 2, 0, 3)).reshape(5 * C, 5 * C).astype(_BF)
    bpf = (bp + b3 @ wp[:C, :] + b5 @ wp[C:, :]).reshape(C, 1)

    f2wt = f2w.T
    f2br = f2b.reshape(C, 1)
    bns2 = (0.5 * bns).reshape(C, 1)
    bnb2 = bnb.reshape(C, 1)

    dd = jnp.arange(-2, 3)[:, None]
    hh = jnp.arange(H)[None, :]
    ww = jnp.arange(W)[None, :]
    hv = (hh + dd >= 0) & (hh + dd < H)
    wv = (ww + dd >= 0) & (ww + dd < W)
    rowm = (hv.astype(jnp.int32) * jnp.int32(-1))[:, :, None]
    rowm = jnp.broadcast_to(rowm, (5, H, W)).reshape(5, HW)
    colm = (wv.astype(jnp.int32) * jnp.int32(-1))[:, None, :]
    colm = jnp.broadcast_to(colm, (5, H, W)).reshape(5, HW)

    weights = (rowm, colm, w1s, b1r, mks, bpf, f1w, f1b, f2wt, f2br, bns2, bnb2)

    def const_spec(a):
        nd = a.ndim
        return pl.BlockSpec(a.shape, lambda b, _nd=nd: (0,) * _nd)

    in_specs = [pl.BlockSpec((BT, C, HW), lambda b: (b, 0, 0))]
    in_specs += [const_spec(a) for a in weights]

    out2 = pl.pallas_call(
        _make_body(H, W, C, BT),
        out_shape=jax.ShapeDtypeStruct((B, C, HW), jnp.float32),
        grid_spec=pltpu.PrefetchScalarGridSpec(
            num_scalar_prefetch=0,
            grid=(B // BT,),
            in_specs=in_specs,
            out_specs=pl.BlockSpec((BT, C, HW), lambda b: (b, 0, 0)),
            scratch_shapes=[
                pltpu.VMEM((3 * C, BT * HW), _BF),
                pltpu.VMEM((5 * C, BT * HW), _BF),
            ],
        ),
        compiler_params=pltpu.CompilerParams(
            dimension_semantics=("parallel",),
            vmem_limit_bytes=100 * 1024 * 1024,
        ),
    )(x2, *weights)
    return out2.reshape(B, C, H, W)


# speedup vs baseline: 3.4508x; 1.0071x over previous
import jax
import jax.numpy as jnp
from jax.experimental import pallas as pl
from jax.experimental.pallas import tpu as pltpu

_SQRT_2_OVER_PI = 0.7978845608028654
_BF = jnp.bfloat16


def _make_body(H, W, C, BT):
    HW = H * W

    def body(x_ref, rowm_ref, colm_ref, w1s_ref, b1_ref, mks_ref, bpf_ref,
             f1w_ref, f1b_ref, f2wt_ref, f2b_ref, bns_ref, bnb_ref, out_ref,
             x3_ref, x5_ref):

        def row_shifted(ai32, dy):
            if dy == 0:
                return ai32
            s = pltpu.roll(ai32, (-dy * W) % HW, axis=1)
            return s & rowm_ref[dy + 2:dy + 3, :]

        def to_i32(a_bf):
            return pltpu.bitcast(a_bf, jnp.int32)

        def to_bf(a_i32):
            return pltpu.bitcast(a_i32, _BF)

        def col_shifted(g_f32, dx):
            s = pltpu.roll(to_i32(g_f32.astype(_BF)), (-dx) % HW, axis=1)
            return to_bf(s & colm_ref[dx + 2:dx + 3, :])

        for i in range(BT):
            xi = to_i32(x_ref[i].astype(_BF))
            for j in range(3):
                x3_ref[j * C:(j + 1) * C, i * HW:(i + 1) * HW] = (
                    to_bf(row_shifted(xi, j - 1)))
        a = jnp.dot(w1s_ref[...], x3_ref[...],
                    preferred_element_type=jnp.float32)

        for i in range(BT):
            sl = slice(i * HW, (i + 1) * HW)
            y1 = (a[C:2 * C, sl] + b1_ref[...]
                  + (col_shifted(a[:C, sl], -1)
                     + col_shifted(a[2 * C:, sl], 1)).astype(jnp.float32))
            yi = to_i32(y1.astype(_BF))
            for j in range(5):
                x5_ref[j * C:(j + 1) * C, i * HW:(i + 1) * HW] = (
                    to_bf(row_shifted(yi, j - 2)))
        g = jnp.dot(mks_ref[...], x5_ref[...],
                    preferred_element_type=jnp.float32)

        for i in range(BT):
            sl = slice(i * HW, (i + 1) * HW)
            zs = ((col_shifted(g[:C, sl], -2)
                   + col_shifted(g[C:2 * C, sl], -1))
                  + (col_shifted(g[3 * C:4 * C, sl], 1)
                     + col_shifted(g[4 * C:, sl], 2)))
            z = g[2 * C:3 * C, sl] + bpf_ref[...] + zs.astype(jnp.float32)

            gap = jnp.sum(z, axis=1, keepdims=True) * (1.0 / HW)
            h = jnp.maximum(
                jnp.sum(f1w_ref[...] * gap, axis=0, keepdims=True)
                + f1b_ref[...], 0.0)
            gate = jax.nn.sigmoid(
                jnp.sum(f2wt_ref[...] * h, axis=1, keepdims=True)
                + f2b_ref[...])
            zg = z * gate

            inner = _SQRT_2_OVER_PI * (zg + 0.044715 * (zg * zg * zg))
            act = zg * (1.0 + jnp.tanh(inner))
            out_ref[i] = act * bns_ref[...] + bnb_ref[...] + x_ref[i]

    return body


def kernel(x, w1, b1, w3, b3, w5, b5, wp, bp, f1w, f1b, f2w, f2b, bns, bnb):
    B, C, H, W = x.shape
    HW = H * W
    BT = 8 if B % 8 == 0 else (4 if B % 4 == 0 else (2 if B % 2 == 0 else 1))
    x2 = x.reshape(B, C, HW).astype(jnp.float32)

    w1r = w1.reshape(3, 3, C, C)
    w1s = jnp.transpose(w1r, (1, 3, 0, 2)).reshape(3 * C, 3 * C).astype(_BF)
    b1r = b1.reshape(C, 1)

    wp3t = wp[:C, :].T
    wp5t = wp[C:, :].T
    w3p = jnp.zeros((5, 5, C), w3.dtype).at[1:4, 1:4, :].set(w3.reshape(3, 3, C))
    mk4 = (wp5t[None, None] * w5.reshape(5, 5, C)[:, :, None, :]
           + wp3t[None, None] * w3p[:, :, None, :])
    mks = jnp.transpose(mk4, (1, 2, 0, 3)).reshape(5 * C, 5 * C).astype(_BF)
    bpf = (bp + b3 @ wp[:C, :] + b5 @ wp[C:, :]).reshape(C, 1)

    f2wt = f2w.T
    f2br = f2b.reshape(C, 1)
    bns2 = (0.5 * bns).reshape(C, 1)
    bnb2 = bnb.reshape(C, 1)

    dd = jnp.arange(-2, 3)[:, None]
    hh = jnp.arange(H)[None, :]
    ww = jnp.arange(W)[None, :]
    hv = (hh + dd >= 0) & (hh + dd < H)
    wv = (ww + dd >= 0) & (ww + dd < W)
    rowm = (hv.astype(jnp.int32) * jnp.int32(-1))[:, :, None]
    rowm = jnp.broadcast_to(rowm, (5, H, W)).reshape(5, HW)
    colm = (wv.astype(jnp.int32) * jnp.int32(-1))[:, None, :]
    colm = jnp.broadcast_to(colm, (5, H, W)).reshape(5, HW)

    weights = (rowm, colm, w1s, b1r, mks, bpf, f1w, f1b, f2wt, f2br, bns2, bnb2)

    def const_spec(a):
        nd = a.ndim
        return pl.BlockSpec(a.shape, lambda b, _nd=nd: (0,) * _nd)

    in_specs = [pl.BlockSpec((BT, C, HW), lambda b: (b, 0, 0))]
    in_specs += [const_spec(a) for a in weights]

    out2 = pl.pallas_call(
        _make_body(H, W, C, BT),
        out_shape=jax.ShapeDtypeStruct((B, C, HW), jnp.float32),
        grid_spec=pltpu.PrefetchScalarGridSpec(
            num_scalar_prefetch=0,
            grid=(B // BT,),
            in_specs=in_specs,
            out_specs=pl.BlockSpec((BT, C, HW), lambda b: (b, 0, 0)),
            scratch_shapes=[
                pltpu.VMEM((3 * C, BT * HW), _BF),
                pltpu.VMEM((5 * C, BT * HW), _BF),
            ],
        ),
        compiler_params=pltpu.CompilerParams(
            dimension_semantics=("parallel",),
            vmem_limit_bytes=100 * 1024 * 1024,
        ),
    )(x2, *weights)
    return out2.reshape(B, C, H, W)
